# baseline (speedup 1.0000x reference)
import jax
import jax.numpy as jnp
from jax.experimental import pallas as pl
from jax.experimental.pallas import tpu as pltpu

N = 4096
C = 32
K = 5
BR = 256
NB = N // BR


def _dot(a, b):
    return jax.lax.dot_general(
        a, b, (((1,), (0,)), ((), ())), preferred_element_type=jnp.float32
    )


def _bf(t):
    return t.astype(jnp.bfloat16)


_CA = 0
_CYD = 32
_CYU = 40
_CGD = 48
_CG = 56
_CTU = 64
_COU = 72
_CTD = 80


def _scnn_body(x_ref, d_ref, u_ref, uw_ref, lb_ref, o_ref, s_ref, p_ref):
    i = pl.program_id(0)
    r = jax.lax.rem(i, NB)
    rb = pl.ds(r * BR, BR)

    @pl.when(i == 0)
    def _init():
        p_ref[:, _CA:_CA + K * K] = _dot(x_ref[...], uw_ref[...])

    @pl.when(i < NB)
    def _p0():
        bb = _bf(d_ref[...])
        s_ref[rb, :] = bb
        p_ref[rb, _CYD:_CYD + K] = _dot(
            bb, _bf(p_ref[:, _CA + 2 * K:_CA + 3 * K]))

    @pl.when(jnp.logical_and(i >= NB, i < 2 * NB))
    def _p1():
        rhs = _bf(p_ref[:, _CA + K:_CA + 2 * K] + p_ref[:, _CYD:_CYD + K])
        p_ref[rb, _CGD:_CGD + K] = _dot(s_ref[rb, :], rhs)
        bb = _bf(u_ref[...])
        s_ref[rb, :] = bb
        p_ref[rb, _CYU:_CYU + K] = _dot(
            bb, _bf(p_ref[:, _CA + 4 * K:_CA + 5 * K]))

    @pl.when(jnp.logical_and(i >= 2 * NB, i < 3 * NB))
    def _p2():
        rhs = _bf(p_ref[:, _CA + 3 * K:_CA + 4 * K] + p_ref[:, _CYU:_CYU + K])
        gu = _dot(s_ref[rb, :], rhs)
        p_ref[rb, _CG:_CG + K] = (
            p_ref[rb, _CA:_CA + K] + p_ref[rb, _CGD:_CGD + K] + gu)

    @pl.when(jnp.logical_and(i >= 3 * NB, i < 4 * NB))
    def _p3():
        p_ref[rb, _CTU:_CTU + 1] = _dot(
            s_ref[rb, :], _bf(p_ref[:, _CG + 4:_CG + 5]))

    @pl.when(jnp.logical_and(i >= 4 * NB, i < 5 * NB))
    def _p4():
        rhs = _bf(p_ref[:, _CG + 3:_CG + 4] + p_ref[:, _CTU:_CTU + 1])
        p_ref[rb, _COU:_COU + 1] = _dot(s_ref[rb, :], rhs)

    @pl.when(jnp.logical_and(i >= 5 * NB, i < 6 * NB))
    def _p5():
        bb = _bf(d_ref[...])
        s_ref[rb, :] = bb
        p_ref[rb, _CTD:_CTD + 1] = _dot(
            bb, _bf(p_ref[:, _CG + 2:_CG + 3]))

    @pl.when(i >= 6 * NB)
    def _p6():
        rhs = _bf(p_ref[:, _CG + 1:_CG + 2] + p_ref[:, _CTD:_CTD + 1])
        od = _dot(s_ref[rb, :], rhs)
        o_ref[...] = (p_ref[rb, _CG:_CG + 1] + p_ref[rb, _COU:_COU + 1]
                      + od + lb_ref[...])


def _dmap(i):
    return (jnp.where(i >= 5 * NB, jnp.minimum(i - 5 * NB, NB - 1),
                      jnp.minimum(i, NB - 1)), 0)


def _umap(i):
    return (jnp.where(i < NB, 0, jnp.minimum(i - NB, NB - 1)), 0)


def _omap(i):
    return (jnp.maximum(i - 6 * NB, 0), 0)


def kernel(x, laplacian_down, laplacian_up, W0, W1, linear_w, linear_b):
    v = jnp.einsum('iok,ol->ik', W1, linear_w)
    uw = jnp.reshape(jnp.einsum('ioj,ok->ijk', W0, v), (C, K * K))
    lb2 = jnp.reshape(linear_b, (1, 1))
    out = pl.pallas_call(
        _scnn_body,
        grid=(7 * NB,),
        in_specs=[
            pl.BlockSpec((N, C), lambda i: (0, 0)),
            pl.BlockSpec((BR, N), _dmap),
            pl.BlockSpec((BR, N), _umap),
            pl.BlockSpec((C, K * K), lambda i: (0, 0)),
            pl.BlockSpec((1, 1), lambda i: (0, 0)),
        ],
        out_specs=pl.BlockSpec((BR, 1), _omap),
        out_shape=jax.ShapeDtypeStruct((N, 1), jnp.float32),
        scratch_shapes=[
            pltpu.VMEM((N, N), jnp.bfloat16),
            pltpu.VMEM((N, 128), jnp.float32),
        ],
        compiler_params=pltpu.CompilerParams(
            dimension_semantics=("arbitrary",),
        ),
    )(x, laplacian_down, laplacian_up, uw, lb2)
    return out


# speedup vs baseline: 1.4763x; 1.4763x over previous
import jax
import jax.numpy as jnp
from jax.experimental import pallas as pl
from jax.experimental.pallas import tpu as pltpu

N = 4096
C = 32
K = 5
BR = 256
NB = N // BR

_PA0 = 0
_PGD = 8
_PG0 = 16
_POU = 24


def _dot(a, b):
    return jax.lax.dot_general(
        a, b, (((1,), (0,)), ((), ())), preferred_element_type=jnp.float32
    )


def _bf(t):
    return t.astype(jnp.bfloat16)


def _scnn_body(x_ref, d_ref, u_ref, uw_ref, lb_ref, o_ref,
               s_ref, p_ref, r3_ref, r4_ref, r5_ref, b1_ref, b2_ref, b3_ref):
    i = pl.program_id(0)
    r = jax.lax.rem(i, NB)
    rb = pl.ds(r * BR, BR)

    @pl.when(i == 0)
    def _init():
        x = x_ref[...]
        uw = uw_ref[...]
        b1_ref[:, 0:K] = _bf(_dot(x, uw[:, 2 * K:3 * K]))
        b2_ref[:, 0:K] = _bf(_dot(x, uw[:, 4 * K:5 * K]))
        r3_ref[:, 0:K] = _dot(x, uw[:, K:2 * K])
        r4_ref[:, 0:K] = _dot(x, uw[:, 3 * K:4 * K])
        p_ref[:, _PA0:_PA0 + K] = _dot(x, uw[:, 0:K])

    @pl.when(i < NB)
    def _p0():
        bb = _bf(d_ref[...])
        s_ref[rb, :] = bb
        r3_ref[rb, 0:K] = r3_ref[rb, 0:K] + _dot(bb, b1_ref[:, 0:K])

    @pl.when(jnp.logical_and(i >= NB, i < 2 * NB))
    def _p1():
        @pl.when(i == NB)
        def _():
            b1_ref[:, 0:K] = _bf(r3_ref[:, 0:K])
        p_ref[rb, _PGD:_PGD + K] = _dot(s_ref[rb, :], b1_ref[:, 0:K])
        bb = _bf(u_ref[...])
        s_ref[rb, :] = bb
        r4_ref[rb, 0:K] = r4_ref[rb, 0:K] + _dot(bb, b2_ref[:, 0:K])

    @pl.when(jnp.logical_and(i >= 2 * NB, i < 3 * NB))
    def _p2():
        @pl.when(i == 2 * NB)
        def _():
            b2_ref[:, 0:K] = _bf(r4_ref[:, 0:K])
        gu = _dot(s_ref[rb, :], b2_ref[:, 0:K])
        g = p_ref[rb, _PA0:_PA0 + K] + p_ref[rb, _PGD:_PGD + K] + gu
        p_ref[rb, _PG0:_PG0 + 1] = g[:, 0:1]
        r5_ref[rb, 0:1] = g[:, 1:2]
        b3_ref[rb, 0:1] = _bf(g[:, 2:3])
        r3_ref[rb, 0:1] = g[:, 3:4]
        b1_ref[rb, 0:1] = _bf(g[:, 4:5])

    @pl.when(jnp.logical_and(i >= 3 * NB, i < 4 * NB))
    def _p3():
        r3_ref[rb, 0:1] = r3_ref[rb, 0:1] + _dot(s_ref[rb, :], b1_ref[:, 0:1])

    @pl.when(jnp.logical_and(i >= 4 * NB, i < 5 * NB))
    def _p4():
        @pl.when(i == 4 * NB)
        def _():
            b2_ref[:, 0:1] = _bf(r3_ref[:, 0:1])
        p_ref[rb, _POU:_POU + 1] = _dot(s_ref[rb, :], b2_ref[:, 0:1])
        bb = _bf(d_ref[...])
        s_ref[rb, :] = bb
        r5_ref[rb, 0:1] = r5_ref[rb, 0:1] + _dot(bb, b3_ref[:, 0:1])

    @pl.when(i >= 5 * NB)
    def _p5():
        @pl.when(i == 5 * NB)
        def _():
            b1_ref[:, 0:1] = _bf(r5_ref[:, 0:1])
        od = _dot(s_ref[rb, :], b1_ref[:, 0:1])
        o_ref[...] = (p_ref[rb, _PG0:_PG0 + 1] + p_ref[rb, _POU:_POU + 1]
                      + od + lb_ref[...])


def _dmap(i):
    return (jnp.where(i >= 4 * NB, jnp.minimum(i - 4 * NB, NB - 1),
                      jnp.minimum(i, NB - 1)), 0)


def _umap(i):
    return (jnp.where(i < NB, 0, jnp.minimum(i - NB, NB - 1)), 0)


def _omap(i):
    return (jnp.maximum(i - 5 * NB, 0), 0)


def kernel(x, laplacian_down, laplacian_up, W0, W1, linear_w, linear_b):
    v = jnp.einsum('iok,ol->ik', W1, linear_w)
    uw = jnp.reshape(jnp.einsum('ioj,ok->ijk', W0, v), (C, K * K))
    lb2 = jnp.reshape(linear_b, (1, 1))
    out = pl.pallas_call(
        _scnn_body,
        grid=(6 * NB,),
        in_specs=[
            pl.BlockSpec((N, C), lambda i: (0, 0)),
            pl.BlockSpec((BR, N), _dmap),
            pl.BlockSpec((BR, N), _umap),
            pl.BlockSpec((C, K * K), lambda i: (0, 0)),
            pl.BlockSpec((1, 1), lambda i: (0, 0)),
        ],
        out_specs=pl.BlockSpec((BR, 1), _omap),
        out_shape=jax.ShapeDtypeStruct((N, 1), jnp.float32),
        scratch_shapes=[
            pltpu.VMEM((N, N), jnp.bfloat16),
            pltpu.VMEM((N, 128), jnp.float32),
            pltpu.VMEM((N, 128), jnp.float32),
            pltpu.VMEM((N, 128), jnp.float32),
            pltpu.VMEM((N, 128), jnp.float32),
            pltpu.VMEM((N, 128), jnp.bfloat16),
            pltpu.VMEM((N, 128), jnp.bfloat16),
            pltpu.VMEM((N, 128), jnp.bfloat16),
        ],
        compiler_params=pltpu.CompilerParams(
            dimension_semantics=("arbitrary",),
            vmem_limit_bytes=66_000_000,
        ),
    )(x, laplacian_down, laplacian_up, uw, lb2)
    return out


# speedup vs baseline: 1.5856x; 1.0740x over previous
import jax
import jax.numpy as jnp
from jax.experimental import pallas as pl
from jax.experimental.pallas import tpu as pltpu

N = 4096
C = 32
K = 5
BR = 256
NB = N // BR
BRC = 512
NC = N // BRC
_P2 = 2 * NB
_P3 = _P2 + NC
_P4 = _P3 + NC
_P5 = _P4 + NB
_END = _P5 + NC

_PA0 = 0
_PGD = 8
_PG0 = 16
_POU = 24


def _dot(a, b):
    return jax.lax.dot_general(
        a, b, (((1,), (0,)), ((), ())), preferred_element_type=jnp.float32
    )


def _bf(t):
    return t.astype(jnp.bfloat16)


def _scnn_body(x_ref, d_ref, u_ref, uw_ref, lb_ref, o_ref,
               s_ref, p_ref, r3_ref, r4_ref, r5_ref, b1_ref, b2_ref, b3_ref):
    i = pl.program_id(0)
    r = jax.lax.rem(i, NB)
    rb = pl.ds(r * BR, BR)

    @pl.when(i == 0)
    def _init():
        x = x_ref[...]
        uw = uw_ref[...]
        b1_ref[:, 0:K] = _bf(_dot(x, uw[:, 2 * K:3 * K]))
        b2_ref[:, 0:K] = _bf(_dot(x, uw[:, 4 * K:5 * K]))
        r3_ref[:, 0:K] = _dot(x, uw[:, K:2 * K])
        r4_ref[:, 0:K] = _dot(x, uw[:, 3 * K:4 * K])
        p_ref[:, _PA0:_PA0 + K] = _dot(x, uw[:, 0:K])

    @pl.when(i < NB)
    def _p0():
        bb = _bf(d_ref[...])
        s_ref[rb, :] = bb
        r3_ref[rb, 0:K] = r3_ref[rb, 0:K] + _dot(bb, b1_ref[:, 0:K])

    @pl.when(jnp.logical_and(i >= NB, i < 2 * NB))
    def _p1():
        @pl.when(i == NB)
        def _():
            b1_ref[:, 0:K] = _bf(r3_ref[:, 0:K])
        p_ref[rb, _PGD:_PGD + K] = _dot(s_ref[rb, :], b1_ref[:, 0:K])
        bb = _bf(u_ref[...])
        s_ref[rb, :] = bb
        r4_ref[rb, 0:K] = r4_ref[rb, 0:K] + _dot(bb, b2_ref[:, 0:K])

    rc = jax.lax.rem(i, NC)
    cb = pl.ds(rc * BRC, BRC)

    @pl.when(jnp.logical_and(i >= _P2, i < _P3))
    def _p2():
        @pl.when(i == _P2)
        def _():
            b2_ref[:, 0:K] = _bf(r4_ref[:, 0:K])
        gu = _dot(s_ref[cb, :], b2_ref[:, 0:K])
        g = p_ref[cb, _PA0:_PA0 + K] + p_ref[cb, _PGD:_PGD + K] + gu
        p_ref[cb, _PG0:_PG0 + 1] = g[:, 0:1]
        r5_ref[cb, 0:1] = g[:, 1:2]
        b3_ref[cb, 0:1] = _bf(g[:, 2:3])
        r3_ref[cb, 0:1] = g[:, 3:4]
        b1_ref[cb, 0:1] = _bf(g[:, 4:5])

    @pl.when(jnp.logical_and(i >= _P3, i < _P4))
    def _p3():
        r3_ref[cb, 0:1] = r3_ref[cb, 0:1] + _dot(s_ref[cb, :], b1_ref[:, 0:1])

    @pl.when(jnp.logical_and(i >= _P4, i < _P5))
    def _p4():
        @pl.when(i == _P4)
        def _():
            b2_ref[:, 0:1] = _bf(r3_ref[:, 0:1])
        p_ref[rb, _POU:_POU + 1] = _dot(s_ref[rb, :], b2_ref[:, 0:1])
        bb = _bf(d_ref[...])
        s_ref[rb, :] = bb
        r5_ref[rb, 0:1] = r5_ref[rb, 0:1] + _dot(bb, b3_ref[:, 0:1])

    @pl.when(i >= _P5)
    def _p5():
        @pl.when(i == _P5)
        def _():
            b1_ref[:, 0:1] = _bf(r5_ref[:, 0:1])
        od = _dot(s_ref[cb, :], b1_ref[:, 0:1])
        o_ref[...] = (p_ref[cb, _PG0:_PG0 + 1] + p_ref[cb, _POU:_POU + 1]
                      + od + lb_ref[...])


def _dmap(i):
    return (jnp.where(i >= _P4, jnp.minimum(i - _P4, NB - 1),
                      jnp.minimum(i, NB - 1)), 0)


def _umap(i):
    return (jnp.where(i < NB, 0, jnp.minimum(i - NB, NB - 1)), 0)


def _omap(i):
    return (jnp.maximum(i - _P5, 0), 0)


def kernel(x, laplacian_down, laplacian_up, W0, W1, linear_w, linear_b):
    v = jnp.einsum('iok,ol->ik', W1, linear_w)
    uw = jnp.reshape(jnp.einsum('ioj,ok->ijk', W0, v), (C, K * K))
    lb2 = jnp.reshape(linear_b, (1, 1))
    out = pl.pallas_call(
        _scnn_body,
        grid=(_END,),
        in_specs=[
            pl.BlockSpec((N, C), lambda i: (0, 0)),
            pl.BlockSpec((BR, N), _dmap),
            pl.BlockSpec((BR, N), _umap),
            pl.BlockSpec((C, K * K), lambda i: (0, 0)),
            pl.BlockSpec((1, 1), lambda i: (0, 0)),
        ],
        out_specs=pl.BlockSpec((BRC, 1), _omap),
        out_shape=jax.ShapeDtypeStruct((N, 1), jnp.float32),
        scratch_shapes=[
            pltpu.VMEM((N, N), jnp.bfloat16),
            pltpu.VMEM((N, 128), jnp.float32),
            pltpu.VMEM((N, 128), jnp.float32),
            pltpu.VMEM((N, 128), jnp.float32),
            pltpu.VMEM((N, 128), jnp.float32),
            pltpu.VMEM((N, 128), jnp.bfloat16),
            pltpu.VMEM((N, 128), jnp.bfloat16),
            pltpu.VMEM((N, 128), jnp.bfloat16),
        ],
        compiler_params=pltpu.CompilerParams(
            dimension_semantics=("arbitrary",),
            vmem_limit_bytes=66_000_000,
        ),
    )(x, laplacian_down, laplacian_up, uw, lb2)
    return out
